# Initial kernel scaffold; baseline (speedup 1.0000x reference)
import jax
import jax.numpy as jnp
from jax import lax
from jax.experimental import pallas as pl
from jax.experimental.pallas import tpu as pltpu
from jax.experimental.pallas import tpu_sc as plsc

_N = 10000
_E = 320000
_NCORES = 2
_NSUB = 16
_CHUNK = 128
_NPAD = 10240
_EPAD = 327680
_ROWS_PER_TILE = _NPAD // _NSUB
_EDGES_PER_W = _EPAD // (_NCORES * _NSUB)
_CHUNKS_PER_W = _EDGES_PER_W // _CHUNK


def _seg_sum(table, src_idx, dst_idx, d, with_count):
    mesh = plsc.VectorSubcoreMesh(core_axis_name="c", subcore_axis_name="s")
    out_type = [jax.ShapeDtypeStruct((_NCORES, _NPAD, d), jnp.float32)]
    scratch = [
        pltpu.VMEM((_CHUNK,), jnp.int32),
        pltpu.VMEM((_CHUNK,), jnp.int32),
        pltpu.VMEM((_CHUNK, d), jnp.float32),
        pltpu.VMEM_SHARED((_NPAD, d), jnp.float32),
    ]
    if with_count:
        out_type.append(jax.ShapeDtypeStruct((_NCORES, _NPAD, 16), jnp.float32))
        scratch += [
            pltpu.VMEM((_CHUNK, 16), jnp.float32),
            pltpu.VMEM_SHARED((_NPAD, 16), jnp.float32),
        ]

    def body(table_hbm, src_hbm, dst_hbm, out_hbm, *rest):
        if with_count:
            cnt_hbm, src_v, dst_v, rows_v, acc_sh, onz_v, cnt_sh = rest
        else:
            src_v, dst_v, rows_v, acc_sh = rest
        c = lax.axis_index("c")
        s = lax.axis_index("s")
        rbase = s * _ROWS_PER_TILE

        @pl.loop(0, _CHUNK)
        def _(i):
            @pl.loop(0, d, step=16)
            def _(j):
                rows_v[i, pl.ds(j, 16)] = jnp.zeros((16,), jnp.float32)

        @pl.loop(0, _ROWS_PER_TILE // _CHUNK)
        def _(k):
            pltpu.sync_copy(rows_v, acc_sh.at[pl.ds(rbase + k * _CHUNK, _CHUNK)])

        if with_count:
            @pl.loop(0, _CHUNK)
            def _(i):
                onz_v[i, pl.ds(0, 16)] = jnp.zeros((16,), jnp.float32)

            @pl.loop(0, _ROWS_PER_TILE // _CHUNK)
            def _(k):
                pltpu.sync_copy(onz_v, cnt_sh.at[pl.ds(rbase + k * _CHUNK, _CHUNK)])

            @pl.loop(0, _CHUNK)
            def _(i):
                onz_v[i, pl.ds(0, 16)] = jnp.ones((16,), jnp.float32)

        plsc.subcore_barrier()

        ebase = (c * _NSUB + s) * _EDGES_PER_W

        @pl.loop(0, _CHUNKS_PER_W)
        def _(k):
            off = ebase + k * _CHUNK
            pltpu.sync_copy(src_hbm.at[pl.ds(off, _CHUNK)], src_v)
            pltpu.sync_copy(dst_hbm.at[pl.ds(off, _CHUNK)], dst_v)
            pltpu.sync_copy(table_hbm.at[src_v], rows_v)
            pltpu.sync_copy(rows_v, acc_sh.at[dst_v], add=True)
            if with_count:
                pltpu.sync_copy(onz_v, cnt_sh.at[dst_v], add=True)

        plsc.subcore_barrier()

        pltpu.sync_copy(acc_sh.at[pl.ds(rbase, _ROWS_PER_TILE)],
                        out_hbm.at[c, pl.ds(rbase, _ROWS_PER_TILE)])
        if with_count:
            pltpu.sync_copy(cnt_sh.at[pl.ds(rbase, _ROWS_PER_TILE)],
                            cnt_hbm.at[c, pl.ds(rbase, _ROWS_PER_TILE)])

    kfn = pl.kernel(body, out_type=tuple(out_type), mesh=mesh,
                    scratch_types=tuple(scratch))
    res = kfn(table, src_idx, dst_idx)
    return res if with_count else res[0]


def _stage_a_body(x_ref, w_ref, a_ref, r_ref):
    p = jnp.dot(x_ref[...], w_ref[...], preferred_element_type=jnp.float32)
    a_ref[...] = p[:, :128]
    r_ref[...] = p[:, 128:]


def _stage_a(x, wcat):
    return pl.pallas_call(
        _stage_a_body,
        grid=(10,),
        in_specs=[pl.BlockSpec((1000, 128), lambda i: (i, 0)),
                  pl.BlockSpec((128, 256), lambda i: (0, 0))],
        out_specs=[pl.BlockSpec((1000, 128), lambda i: (i, 0)),
                   pl.BlockSpec((1000, 128), lambda i: (i, 0))],
        out_shape=[jax.ShapeDtypeStruct((_N, 128), jnp.float32),
                   jax.ShapeDtypeStruct((_N, 128), jnp.float32)],
    )(x, wcat)


def _stage_b_body(p0, p1, c0, c1, xr, b1, wl2, wr2, o1, o2):
    cnt = c0[0][:, :1] + c1[0][:, :1]
    rinv = 1.0 / jnp.maximum(cnt, 1.0)
    mean = (p0[0] + p1[0]) * rinv
    h = jnp.maximum(mean + b1[...] + xr[...], 0.0)
    o1[...] = jnp.dot(h, wl2[...], preferred_element_type=jnp.float32)
    o2[...] = jnp.dot(h, wr2[...], preferred_element_type=jnp.float32)


def _stage_b(parts, cnts, xr1, bl1, wl2p, wr2t):
    return pl.pallas_call(
        _stage_b_body,
        grid=(10,),
        in_specs=[
            pl.BlockSpec((1, 1000, 128), lambda i: (0, i, 0)),
            pl.BlockSpec((1, 1000, 128), lambda i: (1, i, 0)),
            pl.BlockSpec((1, 1000, 16), lambda i: (0, i, 0)),
            pl.BlockSpec((1, 1000, 16), lambda i: (1, i, 0)),
            pl.BlockSpec((1000, 128), lambda i: (i, 0)),
            pl.BlockSpec((1, 128), lambda i: (0, 0)),
            pl.BlockSpec((128, 48), lambda i: (0, 0)),
            pl.BlockSpec((128, 40), lambda i: (0, 0)),
        ],
        out_specs=[pl.BlockSpec((1000, 48), lambda i: (i, 0)),
                   pl.BlockSpec((1000, 40), lambda i: (i, 0))],
        out_shape=[jax.ShapeDtypeStruct((_N, 48), jnp.float32),
                   jax.ShapeDtypeStruct((_N, 40), jnp.float32)],
    )(parts, parts, cnts, cnts, xr1, bl1, wl2p, wr2t)


def _stage_c_body(q0, q1, c0, c1, hr, b2, o):
    cnt = c0[0][:, :1] + c1[0][:, :1]
    rinv = 1.0 / jnp.maximum(cnt, 1.0)
    s2 = (q0[0] + q1[0])[:, :40]
    o[...] = s2 * rinv + b2[...] + hr[...]


def _stage_c(q, cnts, hr2, bl2):
    return pl.pallas_call(
        _stage_c_body,
        grid=(10,),
        in_specs=[
            pl.BlockSpec((1, 1000, 48), lambda i: (0, i, 0)),
            pl.BlockSpec((1, 1000, 48), lambda i: (1, i, 0)),
            pl.BlockSpec((1, 1000, 16), lambda i: (0, i, 0)),
            pl.BlockSpec((1, 1000, 16), lambda i: (1, i, 0)),
            pl.BlockSpec((1000, 40), lambda i: (i, 0)),
            pl.BlockSpec((1, 40), lambda i: (0, 0)),
        ],
        out_specs=pl.BlockSpec((1000, 40), lambda i: (i, 0)),
        out_shape=jax.ShapeDtypeStruct((_N, 40), jnp.float32),
    )(q, q, cnts, cnts, hr2, bl2)


def kernel(x, edge_index, Wl1, bl1, Wr1, Wl2, bl2, Wr2):
    src = edge_index[0].astype(jnp.int32)
    dst = edge_index[1].astype(jnp.int32)
    pad = _EPAD - _E
    src_p = jnp.concatenate([src, jnp.zeros((pad,), jnp.int32)])
    dst_p = jnp.concatenate([dst, jnp.full((pad,), _N, jnp.int32)])

    w1cat = jnp.concatenate([Wl1.T, Wr1.T], axis=1)
    a1, xr1 = _stage_a(x, w1cat)

    parts, cnts = _seg_sum(a1, src_p, dst_p, 128, True)

    wl2p = jnp.pad(Wl2.T, ((0, 0), (0, 8)))
    a2p, hr2 = _stage_b(parts, cnts, xr1, bl1.reshape(1, 128), wl2p, Wr2.T)

    q = _seg_sum(a2p, src_p, dst_p, 48, False)

    return _stage_c(q, cnts, hr2, bl2.reshape(1, 40))


# speedup vs baseline: 2.8455x; 2.8455x over previous
import dataclasses

import jax
import jax.numpy as jnp
from jax import lax
from jax.experimental import pallas as pl
from jax.experimental.pallas import tpu as pltpu
from jax.experimental.pallas import tpu_sc as plsc

_N = 10000
_E = 320000
_NCORES = 2
_NSUB = 16
_NW = _NCORES * _NSUB
_CHUNK = 128
_NPAD = 10240
_EPAD = 327680
_ROWS_PER_TILE = _NPAD // _NSUB
_EDGES_PER_W = _EPAD // _NW
_CHUNKS_PER_W = _EDGES_PER_W // _CHUNK
_RINV_ROWS_PER_TILE = _NPAD // _NW
_WIDE = 64


def _sc_compiler_params():
    cp = pltpu.CompilerParams()
    if "needs_layout_passes" in pltpu.CompilerParams.__dataclass_fields__:
        cp = dataclasses.replace(cp, needs_layout_passes=False)
    return cp


def _seg_sum(table, src_idx, dst_idx, d):
    mesh = plsc.VectorSubcoreMesh(core_axis_name="c", subcore_axis_name="s")

    def body(table_hbm, src_hbm, dst_hbm, out_hbm, src_v, dst_v, rows_v,
             acc_sh):
        c = lax.axis_index("c")
        s = lax.axis_index("s")
        rbase = s * _ROWS_PER_TILE

        @pl.loop(0, _CHUNK)
        def _(i):
            @pl.loop(0, d, step=16)
            def _(j):
                rows_v[i, pl.ds(j, 16)] = jnp.zeros((16,), jnp.float32)

        @pl.loop(0, _ROWS_PER_TILE // _CHUNK)
        def _(k):
            pltpu.sync_copy(rows_v, acc_sh.at[pl.ds(rbase + k * _CHUNK, _CHUNK)])

        plsc.subcore_barrier()

        ebase = (c * _NSUB + s) * _EDGES_PER_W

        @pl.loop(0, _CHUNKS_PER_W)
        def _(k):
            off = ebase + k * _CHUNK
            pltpu.sync_copy(src_hbm.at[pl.ds(off, _CHUNK)], src_v)
            pltpu.sync_copy(dst_hbm.at[pl.ds(off, _CHUNK)], dst_v)
            pltpu.sync_copy(table_hbm.at[src_v], rows_v)
            pltpu.sync_copy(rows_v, acc_sh.at[dst_v], add=True)

        plsc.subcore_barrier()

        obase = c * _NPAD + rbase
        pltpu.sync_copy(acc_sh.at[pl.ds(rbase, _ROWS_PER_TILE)],
                        out_hbm.at[pl.ds(obase, _ROWS_PER_TILE)])

    kfn = pl.kernel(
        body,
        out_type=jax.ShapeDtypeStruct((_NCORES * _NPAD, d), jnp.float32),
        mesh=mesh,
        scratch_types=(
            pltpu.VMEM((_CHUNK,), jnp.int32),
            pltpu.VMEM((_CHUNK,), jnp.int32),
            pltpu.VMEM((_CHUNK, d), jnp.float32),
            pltpu.VMEM_SHARED((_NPAD, d), jnp.float32),
        ))
    return kfn(table, src_idx, dst_idx)


def _count_rinv(dst_idx):
    mesh = plsc.VectorSubcoreMesh(core_axis_name="c", subcore_axis_name="s")

    def body(dst_hbm, rinv_hbm, hist_hbm, dst_v, hist_v, cb_v, ab_v, wide_v):
        c = lax.axis_index("c")
        s = lax.axis_index("s")
        ones16 = jnp.ones((16,), jnp.float32)

        @pl.loop(0, _NPAD, step=16)
        def _(i):
            hist_v[pl.ds(i, 16)] = jnp.zeros((16,), jnp.float32)

        @pl.loop(0, 2)
        def _(h):
            ebase = (s * 2 + h) * _EDGES_PER_W

            @pl.loop(0, _CHUNKS_PER_W)
            def _(k):
                pltpu.sync_copy(dst_hbm.at[pl.ds(ebase + k * _CHUNK, _CHUNK)],
                                dst_v)
                for g in range(_CHUNK // 16):
                    idx16 = dst_v[pl.ds(g * 16, 16)]
                    plsc.addupdate_scatter(hist_v, [idx16], ones16)

        pltpu.sync_copy(hist_v,
                        hist_hbm.at[pl.ds((c * _NSUB + s) * _NPAD, _NPAD)])
        plsc.subcore_barrier()
        gbase = (c * _NSUB + s) * _RINV_ROWS_PER_TILE

        @pl.loop(0, _RINV_ROWS_PER_TILE, step=16)
        def _(i):
            ab_v[pl.ds(i, 16)] = jnp.zeros((16,), jnp.float32)

        @pl.loop(0, _NSUB)
        def _(r):
            pltpu.sync_copy(
                hist_hbm.at[pl.ds((c * _NSUB + r) * _NPAD + gbase,
                                  _RINV_ROWS_PER_TILE)], cb_v)

            @pl.loop(0, _RINV_ROWS_PER_TILE, step=16)
            def _(i):
                ab_v[pl.ds(i, 16)] = ab_v[pl.ds(i, 16)] + cb_v[pl.ds(i, 16)]

        @pl.loop(0, _RINV_ROWS_PER_TILE, step=16)
        def _(i):
            ab_v[pl.ds(i, 16)] = 1.0 / jnp.maximum(ab_v[pl.ds(i, 16)], 1.0)

        @pl.loop(0, _RINV_ROWS_PER_TILE // _WIDE)
        def _(k):
            @pl.loop(0, _WIDE // 16)
            def _(g):
                r16 = ab_v[pl.ds(k * _WIDE + g * 16, 16)]
                for lane in range(16):
                    r = r16[lane]
                    row = (g * 16 + lane) * 128

                    @pl.loop(0, 128, step=16)
                    def _(j):
                        wide_v[pl.ds(row + j, 16)] = jnp.full((16,), r,
                                                              jnp.float32)

            pltpu.sync_copy(
                wide_v,
                rinv_hbm.at[pl.ds((gbase + k * _WIDE) * 128, _WIDE * 128)])

    kfn = pl.kernel(
        body,
        out_type=(jax.ShapeDtypeStruct((_NPAD * 128,), jnp.float32),
                  jax.ShapeDtypeStruct((_NW * _NPAD,), jnp.float32)),
        mesh=mesh,
        compiler_params=_sc_compiler_params(),
        scratch_types=(
            pltpu.VMEM((_CHUNK,), jnp.int32),
            pltpu.VMEM((_NPAD,), jnp.float32),
            pltpu.VMEM((_RINV_ROWS_PER_TILE,), jnp.float32),
            pltpu.VMEM((_RINV_ROWS_PER_TILE,), jnp.float32),
            pltpu.VMEM((_WIDE * 128,), jnp.float32),
        ))
    return kfn(dst_idx)[0]


def _stage_a_body(x_ref, w_ref, a_ref, r_ref):
    p = jnp.dot(x_ref[...], w_ref[...], preferred_element_type=jnp.float32)
    a_ref[...] = p[:, :128]
    r_ref[...] = p[:, 128:]


def _stage_a(x, wcat):
    return pl.pallas_call(
        _stage_a_body,
        grid=(10,),
        in_specs=[pl.BlockSpec((1000, 128), lambda i: (i, 0)),
                  pl.BlockSpec((128, 256), lambda i: (0, 0))],
        out_specs=[pl.BlockSpec((1000, 128), lambda i: (i, 0)),
                   pl.BlockSpec((1000, 128), lambda i: (i, 0))],
        out_shape=[jax.ShapeDtypeStruct((_N, 128), jnp.float32),
                   jax.ShapeDtypeStruct((_N, 128), jnp.float32)],
    )(x, wcat)


def _stage_b_body(p0, p1, rv, xr, b1, wl2, wr2, o1, o2):
    mean = (p0[...] + p1[...]) * rv[...]
    h = jnp.maximum(mean + b1[...] + xr[...], 0.0)
    o1[...] = jnp.dot(h, wl2[...], preferred_element_type=jnp.float32)
    o2[...] = jnp.dot(h, wr2[...], preferred_element_type=jnp.float32)


def _stage_b(parts, rinv, xr1, bl1, wl2p, wr2t):
    return pl.pallas_call(
        _stage_b_body,
        grid=(10,),
        in_specs=[
            pl.BlockSpec((1024, 128), lambda i: (i, 0)),
            pl.BlockSpec((1024, 128), lambda i: (10 + i, 0)),
            pl.BlockSpec((1024, 128), lambda i: (i, 0)),
            pl.BlockSpec((1024, 128), lambda i: (i, 0)),
            pl.BlockSpec((1, 128), lambda i: (0, 0)),
            pl.BlockSpec((128, 128), lambda i: (0, 0)),
            pl.BlockSpec((128, 40), lambda i: (0, 0)),
        ],
        out_specs=[pl.BlockSpec((1024, 128), lambda i: (i, 0)),
                   pl.BlockSpec((1024, 40), lambda i: (i, 0))],
        out_shape=[jax.ShapeDtypeStruct((_N, 128), jnp.float32),
                   jax.ShapeDtypeStruct((_N, 40), jnp.float32)],
    )(parts, parts, rinv, xr1, bl1, wl2p, wr2t)


def _stage_c_body(q0, q1, rv, hr, b2, o):
    s2 = (q0[...] + q1[...])[:, :40]
    o[...] = s2 * rv[:, :40] + b2[...] + hr[...]


def _stage_c(q, rinv, hr2, bl2):
    return pl.pallas_call(
        _stage_c_body,
        grid=(10,),
        in_specs=[
            pl.BlockSpec((1024, 128), lambda i: (i, 0)),
            pl.BlockSpec((1024, 128), lambda i: (10 + i, 0)),
            pl.BlockSpec((1024, 128), lambda i: (i, 0)),
            pl.BlockSpec((1024, 40), lambda i: (i, 0)),
            pl.BlockSpec((1, 40), lambda i: (0, 0)),
        ],
        out_specs=pl.BlockSpec((1024, 40), lambda i: (i, 0)),
        out_shape=jax.ShapeDtypeStruct((_N, 40), jnp.float32),
    )(q, q, rinv, hr2, bl2)


def kernel(x, edge_index, Wl1, bl1, Wr1, Wl2, bl2, Wr2):
    src = edge_index[0].astype(jnp.int32)
    dst = edge_index[1].astype(jnp.int32)
    pad = _EPAD - _E
    src_p = jnp.concatenate([src, jnp.zeros((pad,), jnp.int32)])
    dst_p = jnp.concatenate([dst, jnp.full((pad,), _N, jnp.int32)])

    rinv = _count_rinv(dst_p).reshape(_NPAD, 128)

    w1cat = jnp.concatenate([Wl1.T, Wr1.T], axis=1)
    a1, xr1 = _stage_a(x, w1cat)

    parts = _seg_sum(a1, src_p, dst_p, 128)

    wl2p = jnp.pad(Wl2.T, ((0, 0), (0, 88)))
    a2p, hr2 = _stage_b(parts, rinv, xr1, bl1.reshape(1, 128), wl2p, Wr2.T)

    q = _seg_sum(a2p, src_p, dst_p, 128)

    return _stage_c(q, rinv, hr2, bl2.reshape(1, 40))
